# Initial kernel scaffold; baseline (speedup 1.0000x reference)
import jax
import jax.numpy as jnp
from jax.experimental import pallas as pl


def kernel(xyz1, xyz2):
    raise NotImplementedError("write your pallas kernel here")


# speedup vs baseline: 1.0804x; 1.0804x over previous
import functools

import jax
import jax.numpy as jnp
from jax.experimental import pallas as pl
from jax.experimental.pallas import tpu as pltpu

_S = 8
_L = 128
_BIG = 2**30


def _cd_kernel(x1_ref, x2_ref, d1_ref, i1_ref, d2_ref, i2_ref, cm_ref, cr_ref,
               *, R, C, GR):
    S, L = _S, _L
    M = C * L
    G = R // GR
    lane = jax.lax.broadcasted_iota(jnp.int32, (S, GR), 1)

    cm_ref[...] = jnp.full((S, M), jnp.inf, jnp.float32)
    cr_ref[...] = jnp.zeros((S, M), jnp.int32)

    def group_body(g, carry):
        def strip_body(i, acc):
            accd, acci = acc
            r = g * GR + i
            a = x1_ref[0, r]
            a0 = a[:, 0:1]
            a1 = a[:, 1:2]
            a2 = a[:, 2:3]
            rm = jnp.full((S, L), jnp.inf, jnp.float32)
            cj = jnp.zeros((S, L), jnp.int32)
            for j in range(C):
                sl = slice(j * L, (j + 1) * L)
                b0 = x2_ref[0, 0, sl].reshape(1, L)
                b1 = x2_ref[0, 1, sl].reshape(1, L)
                b2 = x2_ref[0, 2, sl].reshape(1, L)
                t0 = a0 - b0
                t1 = a1 - b1
                t2 = a2 - b2
                d = t0 * t0 + t1 * t1 + t2 * t2
                rmask = d < rm
                rm = jnp.minimum(rm, d)
                cj = jnp.where(rmask, j, cj)
                cmv = cm_ref[:, sl]
                cmask = d < cmv
                cm_ref[:, sl] = jnp.minimum(cmv, d)
                cr_ref[:, sl] = jnp.where(cmask, r, cr_ref[:, sl])
            rowmin = jnp.min(rm, axis=1, keepdims=True)
            lidx = jax.lax.broadcasted_iota(jnp.int32, (S, L), 1)
            marr = cj * L + lidx
            cand = jnp.where(rm == rowmin, marr, _BIG)
            rowidx = jnp.min(cand, axis=1, keepdims=True)
            lm = lane == i
            accd = jnp.where(lm, rowmin, accd)
            acci = jnp.where(lm, rowidx, acci)
            return accd, acci

        accd0 = jnp.zeros((S, GR), jnp.float32)
        acci0 = jnp.zeros((S, GR), jnp.int32)
        accd, acci = jax.lax.fori_loop(0, GR, strip_body, (accd0, acci0))
        d1_ref[0, g] = accd
        i1_ref[0, g] = acci
        return carry

    jax.lax.fori_loop(0, G, group_body, 0)

    cm = cm_ref[...]
    cr = cr_ref[...]
    colmin = jnp.min(cm, axis=0, keepdims=True)
    subM = jax.lax.broadcasted_iota(jnp.int32, (S, M), 0)
    narr = subM * R + cr
    cand2 = jnp.where(cm == colmin, narr, _BIG)
    colidx = jnp.min(cand2, axis=0, keepdims=True)
    d2_ref[0] = colmin
    i2_ref[0] = colidx


@jax.jit
def kernel(xyz1, xyz2):
    B, N, _ = xyz1.shape
    M = xyz2.shape[1]
    S, L = _S, _L
    assert N % S == 0 and M % L == 0
    R = N // S
    C = M // L
    GR = min(128, R)
    assert R % GR == 0
    G = R // GR

    x1g = xyz1.reshape(B, S, R, 3).transpose(0, 2, 1, 3)
    x2t = xyz2.transpose(0, 2, 1)

    body = functools.partial(_cd_kernel, R=R, C=C, GR=GR)
    d1, i1, d2, i2 = pl.pallas_call(
        body,
        grid=(B,),
        in_specs=[
            pl.BlockSpec((1, R, S, 3), lambda b: (b, 0, 0, 0)),
            pl.BlockSpec((1, 3, M), lambda b: (b, 0, 0)),
        ],
        out_specs=[
            pl.BlockSpec((1, G, S, GR), lambda b: (b, 0, 0, 0)),
            pl.BlockSpec((1, G, S, GR), lambda b: (b, 0, 0, 0)),
            pl.BlockSpec((1, 1, M), lambda b: (b, 0, 0)),
            pl.BlockSpec((1, 1, M), lambda b: (b, 0, 0)),
        ],
        out_shape=[
            jax.ShapeDtypeStruct((B, G, S, GR), jnp.float32),
            jax.ShapeDtypeStruct((B, G, S, GR), jnp.int32),
            jax.ShapeDtypeStruct((B, 1, M), jnp.float32),
            jax.ShapeDtypeStruct((B, 1, M), jnp.int32),
        ],
        scratch_shapes=[
            pltpu.VMEM((S, C * L), jnp.float32),
            pltpu.VMEM((S, C * L), jnp.int32),
        ],
    )(x1g, x2t)

    dist1 = d1.transpose(0, 2, 1, 3).reshape(B, N)
    idx1 = i1.transpose(0, 2, 1, 3).reshape(B, N)
    dist2 = d2.reshape(B, M)
    idx2 = i2.reshape(B, M)
    return dist1, dist2, idx1, idx2


# speedup vs baseline: 1.5600x; 1.4440x over previous
import functools

import jax
import jax.numpy as jnp
from jax.experimental import pallas as pl
from jax.experimental.pallas import tpu as pltpu

_S = 8
_L = 128
_BIG = 2**30


def _cd_kernel(x1_ref, x2_ref, d1_ref, i1_ref, d2_ref, i2_ref, cm_ref, cr_ref,
               *, R, C, GR):
    S, L = _S, _L
    M = C * L
    G = R // GR
    NACC = min(4, C)
    CB = -(-C // NACC)
    lane_gr = jax.lax.broadcasted_iota(jnp.int32, (S, GR), 1)
    lane_l = jax.lax.broadcasted_iota(jnp.int32, (S, L), 1)

    cm_ref[...] = jnp.full((S, M), jnp.inf, jnp.float32)
    cr_ref[...] = jnp.zeros((S, M), jnp.int32)

    def strip_compute(r):
        a = x1_ref[0, r]
        a0 = a[:, 0:1]
        a1 = a[:, 1:2]
        a2 = a[:, 2:3]
        rms = [jnp.full((S, L), jnp.inf, jnp.float32) for _ in range(NACC)]
        cjs = [jnp.zeros((S, L), jnp.int32) for _ in range(NACC)]
        for j in range(C):
            k = j // CB
            sl = slice(j * L, (j + 1) * L)
            b0 = x2_ref[0, 0, sl].reshape(1, L)
            b1 = x2_ref[0, 1, sl].reshape(1, L)
            b2 = x2_ref[0, 2, sl].reshape(1, L)
            t0 = a0 - b0
            t1 = a1 - b1
            t2 = a2 - b2
            d = t0 * t0 + t1 * t1 + t2 * t2
            rmask = d < rms[k]
            rms[k] = jnp.minimum(rms[k], d)
            cjs[k] = jnp.where(rmask, j, cjs[k])
            cmv = cm_ref[:, sl]
            cmask = d < cmv
            cm_ref[:, sl] = jnp.minimum(cmv, d)
            cr_ref[:, sl] = jnp.where(cmask, r, cr_ref[:, sl])
        rm, cj = rms[0], cjs[0]
        for k in range(1, NACC):
            mk = rms[k] < rm
            rm = jnp.minimum(rm, rms[k])
            cj = jnp.where(mk, cjs[k], cj)
        return rm, cj

    def reduce_strip(rm, cj, i_prev, accd, acci):
        rowmin = jnp.min(rm, axis=1, keepdims=True)
        marr = cj * L + lane_l
        cand = jnp.where(rm == rowmin, marr, _BIG)
        rowidx = jnp.min(cand, axis=1, keepdims=True)
        lm = lane_gr == i_prev
        rowmin_b = jnp.broadcast_to(rowmin, (S, GR))
        rowidx_b = jnp.broadcast_to(rowidx, (S, GR))
        accd = jnp.where(lm, rowmin_b, accd)
        acci = jnp.where(lm, rowidx_b, acci)
        return accd, acci

    def group_body(g, carry):
        def strip_body(i, acc):
            rm_p, cj_p, accd, acci = acc
            accd, acci = reduce_strip(rm_p, cj_p, i - 1, accd, acci)
            rm, cj = strip_compute(g * GR + i)
            return rm, cj, accd, acci

        init = (cm_ref[:, 0:L],
                cr_ref[:, 0:L],
                cm_ref[:, 0:GR],
                cr_ref[:, 0:GR])
        rm, cj, accd, acci = jax.lax.fori_loop(0, GR, strip_body, init)
        accd, acci = reduce_strip(rm, cj, GR - 1, accd, acci)
        d1_ref[0, g] = accd
        i1_ref[0, g] = acci
        return carry

    jax.lax.fori_loop(0, G, group_body, 0)

    cm = cm_ref[...]
    cr = cr_ref[...]
    colmin = jnp.min(cm, axis=0, keepdims=True)
    subM = jax.lax.broadcasted_iota(jnp.int32, (S, M), 0)
    narr = subM * R + cr
    cand2 = jnp.where(cm == colmin, narr, _BIG)
    colidx = jnp.min(cand2, axis=0, keepdims=True)
    d2_ref[0] = colmin
    i2_ref[0] = colidx


@jax.jit
def kernel(xyz1, xyz2):
    B, N, _ = xyz1.shape
    M = xyz2.shape[1]
    S, L = _S, _L
    assert N % S == 0 and M % L == 0
    R = N // S
    C = M // L
    GR = min(128, R)
    assert R % GR == 0
    G = R // GR

    x1g = xyz1.reshape(B, S, R, 3).transpose(0, 2, 1, 3)
    x2t = xyz2.transpose(0, 2, 1)

    body = functools.partial(_cd_kernel, R=R, C=C, GR=GR)
    d1, i1, d2, i2 = pl.pallas_call(
        body,
        grid=(B,),
        in_specs=[
            pl.BlockSpec((1, R, S, 3), lambda b: (b, 0, 0, 0)),
            pl.BlockSpec((1, 3, M), lambda b: (b, 0, 0)),
        ],
        out_specs=[
            pl.BlockSpec((1, G, S, GR), lambda b: (b, 0, 0, 0)),
            pl.BlockSpec((1, G, S, GR), lambda b: (b, 0, 0, 0)),
            pl.BlockSpec((1, 1, M), lambda b: (b, 0, 0)),
            pl.BlockSpec((1, 1, M), lambda b: (b, 0, 0)),
        ],
        out_shape=[
            jax.ShapeDtypeStruct((B, G, S, GR), jnp.float32),
            jax.ShapeDtypeStruct((B, G, S, GR), jnp.int32),
            jax.ShapeDtypeStruct((B, 1, M), jnp.float32),
            jax.ShapeDtypeStruct((B, 1, M), jnp.int32),
        ],
        scratch_shapes=[
            pltpu.VMEM((S, C * L), jnp.float32),
            pltpu.VMEM((S, C * L), jnp.int32),
        ],
    )(x1g, x2t)

    dist1 = d1.transpose(0, 2, 1, 3).reshape(B, N)
    idx1 = i1.transpose(0, 2, 1, 3).reshape(B, N)
    dist2 = d2.reshape(B, M)
    idx2 = i2.reshape(B, M)
    return dist1, dist2, idx1, idx2
